# baseline (speedup 1.0000x reference)
import functools

import jax
import jax.numpy as jnp
from jax import lax
from jax.experimental import pallas as pl
from jax.experimental.pallas import tpu as pltpu
from jax.experimental.pallas import tpu_sc as plsc

N = 10000
D = 128
HIDDEN = 512
KNN = 32
EPS = 1e-5

NWORK = 32
RW = 320
NP = NWORK * RW
CH = 4
NCH = RW // CH
NV = D // 16
NVB = D // 32
DP = D // 2


def _sc_gather_max(table, idx_flat, *, interpret=False):
    mesh = plsc.VectorSubcoreMesh(
        core_axis_name="c", subcore_axis_name="s", num_cores=2, num_subcores=16)

    @functools.partial(
        pl.kernel,
        out_type=jax.ShapeDtypeStruct((NP, D), jnp.float32),
        mesh=mesh,
        scratch_types=[
            pltpu.VMEM((RW * KNN,), jnp.int32),
            pltpu.VMEM((2, CH * KNN, D), jnp.float32),
            pltpu.VMEM((2, CH, D), jnp.float32),
            pltpu.VMEM_SHARED((N, D), jnp.float32),
            pltpu.SemaphoreType.DMA,
            pltpu.SemaphoreType.DMA,
            pltpu.SemaphoreType.DMA,
            pltpu.SemaphoreType.DMA,
        ],
        interpret=interpret,
    )
    def gmax(table_hbm, idx_hbm, out_hbm, idx_v, rows_v, out_v, shared,
             gsem0, gsem1, osem0, osem1):
        sid = lax.axis_index("s")
        wid = lax.axis_index("c") * 16 + sid
        nbase = wid * RW
        gsems = (gsem0, gsem1)
        rpt = 624
        pltpu.sync_copy(table_hbm.at[pl.ds(sid * rpt, rpt)],
                        shared.at[pl.ds(sid * rpt, rpt)])

        @pl.when(sid == 0)
        def _():
            pltpu.sync_copy(table_hbm.at[pl.ds(16 * rpt, N - 16 * rpt)],
                            shared.at[pl.ds(16 * rpt, N - 16 * rpt)])
        pltpu.sync_copy(idx_hbm.at[pl.ds(nbase * KNN, RW * KNN)], idx_v)
        plsc.subcore_barrier()
        for b in range(2):
            pltpu.async_copy(
                shared.at[idx_v.at[pl.ds(b * (CH * KNN), CH * KNN)]],
                rows_v.at[b], gsems[b])

        osems = (osem0, osem1)

        def compute_chunk(b):
            def node_body(n, _):
                r0 = n * KNN
                for h in range(2):
                    cs = range(h * 4, h * 4 + 4)
                    accs = [rows_v[b, r0, pl.ds(c * 16, 16)] for c in cs]
                    for j in range(1, KNN):
                        for k, c in enumerate(cs):
                            accs[k] = jnp.maximum(
                                accs[k],
                                rows_v[b, r0 + j, pl.ds(c * 16, 16)])
                    for k, c in enumerate(cs):
                        out_v[b, n, pl.ds(c * 16, 16)] = accs[k]
                return 0

            lax.fori_loop(0, CH, node_body, 0)

        @pl.loop(0, NCH, step=2)
        def step(t):
            for b in range(2):
                g = t + b
                pltpu.make_async_copy(
                    shared.at[idx_v.at[pl.ds(g * (CH * KNN), CH * KNN)]],
                    rows_v.at[b], gsems[b]).wait()

                @pl.when(g >= 2)
                def _():
                    pltpu.make_async_copy(
                        out_v.at[b],
                        out_hbm.at[pl.ds(nbase + (g - 2) * CH, CH)],
                        osems[b]).wait()

                compute_chunk(b)
                pltpu.async_copy(
                    out_v.at[b], out_hbm.at[pl.ds(nbase + g * CH, CH)],
                    osems[b])

                @pl.when(g + 2 < NCH)
                def _():
                    pltpu.async_copy(
                        shared.at[idx_v.at[pl.ds((g + 2) * (CH * KNN),
                                                 CH * KNN)]],
                        rows_v.at[b], gsems[b])

        for b in range(2):
            pltpu.make_async_copy(
                out_v.at[b], out_hbm.at[pl.ds(nbase + (NCH - 2 + b) * CH, CH)],
                osems[b]).wait()

    return gmax(table, idx_flat)


BR = 2000
GRID = N // BR


def _row_spec():
    return pl.BlockSpec((BR, D), lambda i: (i, 0))


def _full_spec(shape):
    return pl.BlockSpec(shape, lambda i: tuple(0 for _ in shape))


def _bdot(a, b):
    return jnp.dot(a.astype(jnp.bfloat16), b.astype(jnp.bfloat16),
                   preferred_element_type=jnp.float32)


def _ffn_core(xin, w1, b1, w2, scale, bias):
    h = jnp.maximum(_bdot(xin, w1) + b1, 0.0)
    return _bdot(h, w2) * scale + bias


def _tc_a_body(x_ref, w1, b1, w2, sc2, bi2, wv, bv, x1_ref, v0_ref, v0b_ref):
    x = x_ref[...]
    x1 = x + _ffn_core(x, w1[...], b1[...], w2[...], sc2[...], bi2[...])
    x1_ref[...] = x1
    v0 = _bdot(x1, wv[...]) + bv[...]
    v0_ref[...] = v0
    v0b_ref[...] = v0.astype(jnp.bfloat16)


def _tc_b_body(x1_ref, v0_ref, m0_ref, vs, vb, w1, b1, w2, sc2, bi2, wv, bv,
               x3_ref, v1_ref, v1b_ref):
    m0 = m0_ref[...].astype(jnp.float32)
    x2 = x1_ref[...] + (m0 - v0_ref[...]) * vs[...] + vb[...]
    x3 = x2 + _ffn_core(x2, w1[...], b1[...], w2[...], sc2[...], bi2[...])
    x3_ref[...] = x3
    v1 = _bdot(x3, wv[...]) + bv[...]
    v1_ref[...] = v1
    v1b_ref[...] = v1.astype(jnp.bfloat16)


def _tc_c_body(x3_ref, v1_ref, m1_ref, vs, vb, w1, b1, w2, sc2, bi2, out_ref):
    m1 = m1_ref[...].astype(jnp.float32)
    x4 = x3_ref[...] + (m1 - v1_ref[...]) * vs[...] + vb[...]
    out_ref[...] = x4 + _ffn_core(x4, w1[...], b1[...], w2[...], sc2[...], bi2[...])


def _ffn_prep(p):
    s = 1.0 / jnp.sqrt(1.0 + EPS)
    scale = (p["g"] * s).reshape(1, D)
    bias = (p["l2"]["b"] * p["g"] * s + p["beta"]).reshape(1, D)
    return (p["l1"]["W"], p["l1"]["b"].reshape(1, HIDDEN), p["l2"]["W"], scale, bias)


def _vfr_prep(p):
    s = 1.0 / jnp.sqrt(1.0 + EPS)
    return ((p["g"] * s).reshape(1, D), p["beta"].reshape(1, D),
            p["lin"]["W"], p["lin"]["b"].reshape(1, D))


_FFN_SPECS = [_full_spec((D, HIDDEN)), _full_spec((1, HIDDEN)),
              _full_spec((HIDDEN, D)), _full_spec((1, D)), _full_spec((1, D))]
_VB_SPECS = [_full_spec((1, D)), _full_spec((1, D))]
_LIN_SPECS = [_full_spec((D, D)), _full_spec((1, D))]


def _tc_a(x, ffn, wv, bv, *, interpret=False):
    return pl.pallas_call(
        _tc_a_body,
        grid=(GRID,),
        in_specs=[_row_spec()] + _FFN_SPECS + _LIN_SPECS,
        out_specs=[_row_spec(), _row_spec(), _row_spec()],
        out_shape=[jax.ShapeDtypeStruct((N, D), jnp.float32)] * 2
        + [jax.ShapeDtypeStruct((N, D), jnp.bfloat16)],
        interpret=interpret,
    )(x, *ffn, wv, bv)


def _tc_b(x1, v0, m0, vs, vb, ffn, wv, bv, *, interpret=False):
    return pl.pallas_call(
        _tc_b_body,
        grid=(GRID,),
        in_specs=[_row_spec()] * 3 + _VB_SPECS + _FFN_SPECS + _LIN_SPECS,
        out_specs=[_row_spec(), _row_spec(), _row_spec()],
        out_shape=[jax.ShapeDtypeStruct((N, D), jnp.float32)] * 2
        + [jax.ShapeDtypeStruct((N, D), jnp.bfloat16)],
        interpret=interpret,
    )(x1, v0, m0, vs, vb, *ffn, wv, bv)


def _tc_c(x3, v1, m1, vs, vb, ffn, *, interpret=False):
    return pl.pallas_call(
        _tc_c_body,
        grid=(GRID,),
        in_specs=[_row_spec()] * 3 + _VB_SPECS + _FFN_SPECS,
        out_specs=_row_spec(),
        out_shape=jax.ShapeDtypeStruct((N, D), jnp.float32),
        interpret=interpret,
    )(x3, v1, m1, vs, vb, *ffn)


def kernel(x, knn_idx, params, *, interpret=False, sc_interpret=None):
    if sc_interpret is None:
        sc_interpret = interpret
    x2 = x[0]
    idx = knn_idx[0].reshape(N * KNN)
    idx_pad = jnp.zeros((NP * KNN,), jnp.int32).at[: N * KNN].set(idx)

    mlp0 = _ffn_prep(params["mlp0"])
    ffn0 = _ffn_prep(params["ffn0"])
    ffn1 = _ffn_prep(params["ffn1"])
    vs0, vb0, wv0, bv0 = _vfr_prep(params["vfr0"])
    vs1, vb1, wv1, bv1 = _vfr_prep(params["vfr1"])

    x1, v0, v0b = _tc_a(x2, mlp0, wv0, bv0, interpret=interpret)
    del v0b
    m0 = _sc_gather_max(v0, idx_pad, interpret=sc_interpret)[:N]
    x3, v1, v1b = _tc_b(x1, v0, m0, vs0, vb0, ffn0, wv1, bv1, interpret=interpret)
    del v1b
    m1 = _sc_gather_max(v1, idx_pad, interpret=sc_interpret)[:N]
    out = _tc_c(x3, v1, m1, vs1, vb1, ffn1, interpret=interpret)
    return out[None]


# speedup vs baseline: 1.0123x; 1.0123x over previous
import functools

import jax
import jax.numpy as jnp
from jax import lax
from jax.experimental import pallas as pl
from jax.experimental.pallas import tpu as pltpu
from jax.experimental.pallas import tpu_sc as plsc

N = 10000
D = 128
HIDDEN = 512
KNN = 32
EPS = 1e-5

NWORK = 32
RW = 320
NP = NWORK * RW
CH = 4
NCH = RW // CH
NV = D // 16
NVB = D // 32
DP = D // 2


def _sc_gather_max(table, idx_flat, *, interpret=False):
    mesh = plsc.VectorSubcoreMesh(
        core_axis_name="c", subcore_axis_name="s", num_cores=2, num_subcores=16)

    @functools.partial(
        pl.kernel,
        out_type=jax.ShapeDtypeStruct((NP, D), jnp.float32),
        mesh=mesh,
        scratch_types=[
            pltpu.VMEM((RW * KNN,), jnp.int32),
            pltpu.VMEM((2, CH * KNN, D), jnp.float32),
            pltpu.VMEM((2, CH, D), jnp.float32),
            pltpu.VMEM_SHARED((N, D), jnp.float32),
            pltpu.SemaphoreType.DMA,
            pltpu.SemaphoreType.DMA,
            pltpu.SemaphoreType.DMA,
            pltpu.SemaphoreType.DMA,
        ],
        interpret=interpret,
    )
    def gmax(table_hbm, idx_hbm, out_hbm, idx_v, rows_v, out_v, shared,
             gsem0, gsem1, osem0, osem1):
        sid = lax.axis_index("s")
        wid = lax.axis_index("c") * 16 + sid
        nbase = wid * RW
        gsems = (gsem0, gsem1)
        rpt = 624
        pltpu.sync_copy(table_hbm.at[pl.ds(sid * rpt, rpt)],
                        shared.at[pl.ds(sid * rpt, rpt)])

        @pl.when(sid == 0)
        def _():
            pltpu.sync_copy(table_hbm.at[pl.ds(16 * rpt, N - 16 * rpt)],
                            shared.at[pl.ds(16 * rpt, N - 16 * rpt)])
        pltpu.sync_copy(idx_hbm.at[pl.ds(nbase * KNN, RW * KNN)], idx_v)
        plsc.subcore_barrier()
        for b in range(2):
            pltpu.async_copy(
                shared.at[idx_v.at[pl.ds(b * (CH * KNN), CH * KNN)]],
                rows_v.at[b], gsems[b])

        osems = (osem0, osem1)

        def compute_chunk(b):
            def node_body(n, _):
                r0 = n * KNN
                for h in range(2):
                    cs = range(h * 4, h * 4 + 4)
                    accs = [rows_v[b, r0, pl.ds(c * 16, 16)] for c in cs]
                    for j in range(1, KNN):
                        for k, c in enumerate(cs):
                            accs[k] = jnp.maximum(
                                accs[k],
                                rows_v[b, r0 + j, pl.ds(c * 16, 16)])
                    for k, c in enumerate(cs):
                        out_v[b, n, pl.ds(c * 16, 16)] = accs[k]
                return 0

            lax.fori_loop(0, CH, node_body, 0)

        @pl.loop(0, NCH, step=2)
        def step(t):
            for b in range(2):
                g = t + b
                pltpu.make_async_copy(
                    shared.at[idx_v.at[pl.ds(g * (CH * KNN), CH * KNN)]],
                    rows_v.at[b], gsems[b]).wait()

                @pl.when(g >= 2)
                def _():
                    pltpu.make_async_copy(
                        out_v.at[b],
                        out_hbm.at[pl.ds(nbase + (g - 2) * CH, CH)],
                        osems[b]).wait()

                compute_chunk(b)
                pltpu.async_copy(
                    out_v.at[b], out_hbm.at[pl.ds(nbase + g * CH, CH)],
                    osems[b])

                @pl.when(g + 2 < NCH)
                def _():
                    pltpu.async_copy(
                        shared.at[idx_v.at[pl.ds((g + 2) * (CH * KNN),
                                                 CH * KNN)]],
                        rows_v.at[b], gsems[b])

        for b in range(2):
            pltpu.make_async_copy(
                out_v.at[b], out_hbm.at[pl.ds(nbase + (NCH - 2 + b) * CH, CH)],
                osems[b]).wait()

    return gmax(table, idx_flat)


BR = 2048
GRID = NP // BR


def _row_spec():
    return pl.BlockSpec((BR, D), lambda i: (i, 0))


def _full_spec(shape):
    return pl.BlockSpec(shape, lambda i: tuple(0 for _ in shape))


def _bdot(a, b):
    return jnp.dot(a.astype(jnp.bfloat16), b.astype(jnp.bfloat16),
                   preferred_element_type=jnp.float32)


def _ffn_core(xin, w1, b1, w2, scale, bias):
    h = jnp.maximum(_bdot(xin, w1) + b1, 0.0)
    return _bdot(h, w2) * scale + bias


def _tc_a_body(x_ref, w1, b1, w2, sc2, bi2, wv, bv, x1_ref, v0_ref):
    x = x_ref[...]
    x1 = x + _ffn_core(x, w1[...], b1[...], w2[...], sc2[...], bi2[...])
    x1_ref[...] = x1
    v0_ref[...] = _bdot(x1, wv[...]) + bv[...]


def _tc_b_body(x1_ref, v0_ref, m0_ref, vs, vb, w1, b1, w2, sc2, bi2, wv, bv,
               x3_ref, v1_ref):
    x2 = x1_ref[...] + (m0_ref[...] - v0_ref[...]) * vs[...] + vb[...]
    x3 = x2 + _ffn_core(x2, w1[...], b1[...], w2[...], sc2[...], bi2[...])
    x3_ref[...] = x3
    v1_ref[...] = _bdot(x3, wv[...]) + bv[...]


def _tc_c_body(x3_ref, v1_ref, m1_ref, vs, vb, w1, b1, w2, sc2, bi2, out_ref):
    x4 = x3_ref[...] + (m1_ref[...] - v1_ref[...]) * vs[...] + vb[...]
    out_ref[...] = x4 + _ffn_core(x4, w1[...], b1[...], w2[...], sc2[...], bi2[...])


def _ffn_prep(p):
    s = 1.0 / jnp.sqrt(1.0 + EPS)
    scale = (p["g"] * s).reshape(1, D)
    bias = (p["l2"]["b"] * p["g"] * s + p["beta"]).reshape(1, D)
    return (p["l1"]["W"], p["l1"]["b"].reshape(1, HIDDEN), p["l2"]["W"], scale, bias)


def _vfr_prep(p):
    s = 1.0 / jnp.sqrt(1.0 + EPS)
    return ((p["g"] * s).reshape(1, D), p["beta"].reshape(1, D),
            p["lin"]["W"], p["lin"]["b"].reshape(1, D))


_FFN_SPECS = [_full_spec((D, HIDDEN)), _full_spec((1, HIDDEN)),
              _full_spec((HIDDEN, D)), _full_spec((1, D)), _full_spec((1, D))]
_VB_SPECS = [_full_spec((1, D)), _full_spec((1, D))]
_LIN_SPECS = [_full_spec((D, D)), _full_spec((1, D))]


def _tc_a(x, ffn, wv, bv, *, interpret=False):
    return pl.pallas_call(
        _tc_a_body,
        grid=(GRID,),
        in_specs=[_row_spec()] + _FFN_SPECS + _LIN_SPECS,
        out_specs=[_row_spec(), _row_spec()],
        out_shape=[jax.ShapeDtypeStruct((NP, D), jnp.float32)] * 2,
        interpret=interpret,
    )(x, *ffn, wv, bv)


def _tc_b(x1, v0, m0, vs, vb, ffn, wv, bv, *, interpret=False):
    return pl.pallas_call(
        _tc_b_body,
        grid=(GRID,),
        in_specs=[_row_spec()] * 3 + _VB_SPECS + _FFN_SPECS + _LIN_SPECS,
        out_specs=[_row_spec(), _row_spec()],
        out_shape=[jax.ShapeDtypeStruct((NP, D), jnp.float32)] * 2,
        interpret=interpret,
    )(x1, v0, m0, vs, vb, *ffn, wv, bv)


def _tc_c(x3, v1, m1, vs, vb, ffn, *, interpret=False):
    return pl.pallas_call(
        _tc_c_body,
        grid=(GRID,),
        in_specs=[_row_spec()] * 3 + _VB_SPECS + _FFN_SPECS,
        out_specs=_row_spec(),
        out_shape=jax.ShapeDtypeStruct((NP, D), jnp.float32),
        interpret=interpret,
    )(x3, v1, m1, vs, vb, *ffn)


def kernel(x, knn_idx, params, *, interpret=False, sc_interpret=None):
    if sc_interpret is None:
        sc_interpret = interpret
    x2 = jnp.zeros((NP, D), jnp.float32).at[:N].set(x[0])
    idx = knn_idx[0].reshape(N * KNN)
    idx_pad = jnp.zeros((NP * KNN,), jnp.int32).at[: N * KNN].set(idx)

    mlp0 = _ffn_prep(params["mlp0"])
    ffn0 = _ffn_prep(params["ffn0"])
    ffn1 = _ffn_prep(params["ffn1"])
    vs0, vb0, wv0, bv0 = _vfr_prep(params["vfr0"])
    vs1, vb1, wv1, bv1 = _vfr_prep(params["vfr1"])

    x1, v0 = _tc_a(x2, mlp0, wv0, bv0, interpret=interpret)
    m0 = _sc_gather_max(v0, idx_pad, interpret=sc_interpret)
    x3, v1 = _tc_b(x1, v0, m0, vs0, vb0, ffn0, wv1, bv1, interpret=interpret)
    m1 = _sc_gather_max(v1, idx_pad, interpret=sc_interpret)
    out = _tc_c(x3, v1, m1, vs1, vb1, ffn1, interpret=interpret)
    return out[:N][None]


# speedup vs baseline: 1.0595x; 1.0466x over previous
import functools

import jax
import jax.numpy as jnp
from jax import lax
from jax.experimental import pallas as pl
from jax.experimental.pallas import tpu as pltpu
from jax.experimental.pallas import tpu_sc as plsc

N = 10000
D = 128
HIDDEN = 512
KNN = 32
EPS = 1e-5

NWORK = 32
RW = 320
NP = NWORK * RW
CH = 4
NCH = RW // CH
NV = D // 16
NVB = D // 32
DP = D // 2


def _sc_gather_max(table, idx_flat, *, interpret=False):
    mesh = plsc.VectorSubcoreMesh(
        core_axis_name="c", subcore_axis_name="s", num_cores=2, num_subcores=16)

    @functools.partial(
        pl.kernel,
        out_type=jax.ShapeDtypeStruct((NP, D), jnp.float32),
        mesh=mesh,
        scratch_types=[
            pltpu.VMEM((RW * KNN,), jnp.int32),
            pltpu.VMEM((2, CH * KNN, D), jnp.float32),
            pltpu.VMEM((2, CH, D), jnp.float32),
            pltpu.VMEM_SHARED((N, D), jnp.float32),
            pltpu.SemaphoreType.DMA,
            pltpu.SemaphoreType.DMA,
            pltpu.SemaphoreType.DMA,
            pltpu.SemaphoreType.DMA,
        ],
        interpret=interpret,
    )
    def gmax(table_hbm, idx_hbm, out_hbm, idx_v, rows_v, out_v, shared,
             gsem0, gsem1, osem0, osem1):
        sid = lax.axis_index("s")
        wid = lax.axis_index("c") * 16 + sid
        nbase = wid * RW
        gsems = (gsem0, gsem1)
        rpt = 624
        pltpu.sync_copy(table_hbm.at[pl.ds(sid * rpt, rpt)],
                        shared.at[pl.ds(sid * rpt, rpt)])

        @pl.when(sid == 0)
        def _():
            pltpu.sync_copy(table_hbm.at[pl.ds(16 * rpt, N - 16 * rpt)],
                            shared.at[pl.ds(16 * rpt, N - 16 * rpt)])
        pltpu.sync_copy(idx_hbm.at[pl.ds(nbase * KNN, RW * KNN)], idx_v)
        plsc.subcore_barrier()
        for b in range(2):
            pltpu.async_copy(
                shared.at[idx_v.at[pl.ds(b * (CH * KNN), CH * KNN)]],
                rows_v.at[b], gsems[b])

        osems = (osem0, osem1)

        def compute_chunk(b):
            def node_body(n, _):
                r0 = n * KNN
                for h in range(4):
                    cs = range(h * 2, h * 2 + 2)
                    accs = [rows_v[b, r0, pl.ds(c * 16, 16)] for c in cs]
                    for j in range(1, KNN):
                        for k, c in enumerate(cs):
                            accs[k] = jnp.maximum(
                                accs[k],
                                rows_v[b, r0 + j, pl.ds(c * 16, 16)])
                    for k, c in enumerate(cs):
                        out_v[b, n, pl.ds(c * 16, 16)] = accs[k]
                return 0

            lax.fori_loop(0, CH, node_body, 0)

        @pl.loop(0, NCH, step=2)
        def step(t):
            for b in range(2):
                g = t + b
                pltpu.make_async_copy(
                    shared.at[idx_v.at[pl.ds(g * (CH * KNN), CH * KNN)]],
                    rows_v.at[b], gsems[b]).wait()

                @pl.when(g >= 2)
                def _():
                    pltpu.make_async_copy(
                        out_v.at[b],
                        out_hbm.at[pl.ds(nbase + (g - 2) * CH, CH)],
                        osems[b]).wait()

                compute_chunk(b)
                pltpu.async_copy(
                    out_v.at[b], out_hbm.at[pl.ds(nbase + g * CH, CH)],
                    osems[b])

                @pl.when(g + 2 < NCH)
                def _():
                    pltpu.async_copy(
                        shared.at[idx_v.at[pl.ds((g + 2) * (CH * KNN),
                                                 CH * KNN)]],
                        rows_v.at[b], gsems[b])

        for b in range(2):
            pltpu.make_async_copy(
                out_v.at[b], out_hbm.at[pl.ds(nbase + (NCH - 2 + b) * CH, CH)],
                osems[b]).wait()

    return gmax(table, idx_flat)


BR = 2048
GRID = NP // BR


def _row_spec():
    return pl.BlockSpec((BR, D), lambda i: (i, 0))


def _full_spec(shape):
    return pl.BlockSpec(shape, lambda i: tuple(0 for _ in shape))


def _bdot(a, b):
    return jnp.dot(a.astype(jnp.bfloat16), b.astype(jnp.bfloat16),
                   preferred_element_type=jnp.float32)


def _ffn_core(xin, w1, b1, w2, scale, bias):
    h = jnp.maximum(_bdot(xin, w1) + b1, 0.0)
    return _bdot(h, w2) * scale + bias


def _tc_a_body(x_ref, w1, b1, w2, sc2, bi2, wv, bv, x1_ref, v0_ref):
    x = x_ref[...]
    x1 = x + _ffn_core(x, w1[...], b1[...], w2[...], sc2[...], bi2[...])
    x1_ref[...] = x1
    v0_ref[...] = _bdot(x1, wv[...]) + bv[...]


def _tc_b_body(x1_ref, v0_ref, m0_ref, vs, vb, w1, b1, w2, sc2, bi2, wv, bv,
               x3_ref, v1_ref):
    x2 = x1_ref[...] + (m0_ref[...] - v0_ref[...]) * vs[...] + vb[...]
    x3 = x2 + _ffn_core(x2, w1[...], b1[...], w2[...], sc2[...], bi2[...])
    x3_ref[...] = x3
    v1_ref[...] = _bdot(x3, wv[...]) + bv[...]


def _tc_c_body(x3_ref, v1_ref, m1_ref, vs, vb, w1, b1, w2, sc2, bi2, out_ref):
    x4 = x3_ref[...] + (m1_ref[...] - v1_ref[...]) * vs[...] + vb[...]
    out_ref[...] = x4 + _ffn_core(x4, w1[...], b1[...], w2[...], sc2[...], bi2[...])


def _ffn_prep(p):
    s = 1.0 / jnp.sqrt(1.0 + EPS)
    scale = (p["g"] * s).reshape(1, D)
    bias = (p["l2"]["b"] * p["g"] * s + p["beta"]).reshape(1, D)
    return (p["l1"]["W"], p["l1"]["b"].reshape(1, HIDDEN), p["l2"]["W"], scale, bias)


def _vfr_prep(p):
    s = 1.0 / jnp.sqrt(1.0 + EPS)
    return ((p["g"] * s).reshape(1, D), p["beta"].reshape(1, D),
            p["lin"]["W"], p["lin"]["b"].reshape(1, D))


_FFN_SPECS = [_full_spec((D, HIDDEN)), _full_spec((1, HIDDEN)),
              _full_spec((HIDDEN, D)), _full_spec((1, D)), _full_spec((1, D))]
_VB_SPECS = [_full_spec((1, D)), _full_spec((1, D))]
_LIN_SPECS = [_full_spec((D, D)), _full_spec((1, D))]


def _tc_a(x, ffn, wv, bv, *, interpret=False):
    return pl.pallas_call(
        _tc_a_body,
        grid=(GRID,),
        in_specs=[_row_spec()] + _FFN_SPECS + _LIN_SPECS,
        out_specs=[_row_spec(), _row_spec()],
        out_shape=[jax.ShapeDtypeStruct((NP, D), jnp.float32)] * 2,
        interpret=interpret,
    )(x, *ffn, wv, bv)


def _tc_b(x1, v0, m0, vs, vb, ffn, wv, bv, *, interpret=False):
    return pl.pallas_call(
        _tc_b_body,
        grid=(GRID,),
        in_specs=[_row_spec()] * 3 + _VB_SPECS + _FFN_SPECS + _LIN_SPECS,
        out_specs=[_row_spec(), _row_spec()],
        out_shape=[jax.ShapeDtypeStruct((NP, D), jnp.float32)] * 2,
        interpret=interpret,
    )(x1, v0, m0, vs, vb, *ffn, wv, bv)


def _tc_c(x3, v1, m1, vs, vb, ffn, *, interpret=False):
    return pl.pallas_call(
        _tc_c_body,
        grid=(GRID,),
        in_specs=[_row_spec()] * 3 + _VB_SPECS + _FFN_SPECS,
        out_specs=_row_spec(),
        out_shape=jax.ShapeDtypeStruct((NP, D), jnp.float32),
        interpret=interpret,
    )(x3, v1, m1, vs, vb, *ffn)


def kernel(x, knn_idx, params, *, interpret=False, sc_interpret=None):
    if sc_interpret is None:
        sc_interpret = interpret
    x2 = jnp.zeros((NP, D), jnp.float32).at[:N].set(x[0])
    idx = knn_idx[0].reshape(N * KNN)
    idx_pad = jnp.zeros((NP * KNN,), jnp.int32).at[: N * KNN].set(idx)

    mlp0 = _ffn_prep(params["mlp0"])
    ffn0 = _ffn_prep(params["ffn0"])
    ffn1 = _ffn_prep(params["ffn1"])
    vs0, vb0, wv0, bv0 = _vfr_prep(params["vfr0"])
    vs1, vb1, wv1, bv1 = _vfr_prep(params["vfr1"])

    x1, v0 = _tc_a(x2, mlp0, wv0, bv0, interpret=interpret)
    m0 = _sc_gather_max(v0, idx_pad, interpret=sc_interpret)
    x3, v1 = _tc_b(x1, v0, m0, vs0, vb0, ffn0, wv1, bv1, interpret=interpret)
    m1 = _sc_gather_max(v1, idx_pad, interpret=sc_interpret)
    out = _tc_c(x3, v1, m1, vs1, vb1, ffn1, interpret=interpret)
    return out[:N][None]
